# baseline (speedup 1.0000x reference)
import functools

import jax
import jax.numpy as jnp
from jax import lax
from jax.experimental import pallas as pl
from jax.experimental.pallas import tpu as pltpu
from jax.experimental.pallas import tpu_sc as plsc

SEQ = 4096
CTX = 21
CHAR_V = 1000
POS_V = 627
CHAR_D = 50
POS_D = 20
WORD_D = 100
GAZ_D = 15
EMB = CHAR_D + POS_D + WORD_D + GAZ_D

NC = 2
NS = 16
NW = NC * NS
SEQ_PER_W = SEQ // NW
L = 16
NG = SEQ_PER_W // L


def _sc_gather(sentence, pos, char_table, pos_table):
    mesh = plsc.VectorSubcoreMesh(core_axis_name="c", subcore_axis_name="s")

    @functools.partial(
        pl.kernel,
        out_type=(
            jax.ShapeDtypeStruct((CTX, CHAR_D, SEQ), jnp.float32),
            jax.ShapeDtypeStruct((CTX, POS_D, SEQ), jnp.float32),
        ),
        mesh=mesh,
        compiler_params=pltpu.CompilerParams(needs_layout_passes=False),
        scratch_types=[
            pltpu.VMEM((SEQ_PER_W, CTX), jnp.int32),
            pltpu.VMEM((SEQ_PER_W, CTX), jnp.int32),
            pltpu.VMEM((CTX, SEQ_PER_W), jnp.int32),
            pltpu.VMEM((CTX, SEQ_PER_W), jnp.int32),
            pltpu.VMEM_SHARED((CHAR_V, CHAR_D), jnp.float32),
            pltpu.VMEM_SHARED((POS_V, POS_D), jnp.float32),
            pltpu.VMEM((SEQ_PER_W, CHAR_D), jnp.float32),
            pltpu.VMEM((SEQ_PER_W, POS_D), jnp.float32),
            pltpu.VMEM((CHAR_D, SEQ_PER_W), jnp.float32),
            pltpu.VMEM((POS_D, SEQ_PER_W), jnp.float32),
            pltpu.SemaphoreType.DMA,
            pltpu.SemaphoreType.DMA,
        ],
    )
    def k(sent_hbm, pos_hbm, ctab_hbm, ptab_hbm, gcT_hbm, gpT_hbm,
          idx_c, idx_p, idxT_c, idxT_p, ctab_sh, ptab_sh,
          rowc, rowp, planec, planep, gsem, wsem):
        wid = lax.axis_index("s") * NC + lax.axis_index("c")
        s0 = wid * SEQ_PER_W
        @pl.when(lax.axis_index("s") == 0)
        def _():
            pltpu.sync_copy(ctab_hbm, ctab_sh)
            pltpu.sync_copy(ptab_hbm, ptab_sh)

        pltpu.sync_copy(sent_hbm.at[pl.ds(s0, SEQ_PER_W)], idx_c)
        pltpu.sync_copy(pos_hbm.at[pl.ds(s0, SEQ_PER_W)], idx_p)

        lanes = lax.iota(jnp.int32, L)

        @pl.loop(0, CTX)
        def _(c):
            for g in range(NG):
                rows = lanes + g * L
                col = jnp.full((L,), c, jnp.int32)
                idxT_c.at[c][pl.ds(g * L, L)] = plsc.load_gather(
                    idx_c, [rows, col])
                idxT_p.at[c][pl.ds(g * L, L)] = plsc.load_gather(
                    idx_p, [rows, col])

        plsc.subcore_barrier()

        @pl.loop(0, CTX)
        def _(c):
            pltpu.async_copy(ctab_sh.at[idxT_c.at[c]], rowc, gsem).wait()
            pltpu.async_copy(ptab_sh.at[idxT_p.at[c]], rowp, gsem).wait()

            @pl.when(c > 0)
            def _():
                pltpu.make_async_copy(
                    planec, gcT_hbm.at[c - 1, slice(None),
                                       pl.ds(s0, SEQ_PER_W)], wsem).wait()
                pltpu.make_async_copy(
                    planep, gpT_hbm.at[c - 1, slice(None),
                                       pl.ds(s0, SEQ_PER_W)], wsem).wait()

            @pl.loop(0, CHAR_D)
            def _(d):
                col = jnp.full((L,), d, jnp.int32)
                for g in range(NG):
                    rows = lanes + g * L
                    planec.at[d][pl.ds(g * L, L)] = plsc.load_gather(
                        rowc, [rows, col])

            @pl.loop(0, POS_D)
            def _(d):
                col = jnp.full((L,), d, jnp.int32)
                for g in range(NG):
                    rows = lanes + g * L
                    planep.at[d][pl.ds(g * L, L)] = plsc.load_gather(
                        rowp, [rows, col])

            pltpu.async_copy(
                planec, gcT_hbm.at[c, slice(None), pl.ds(s0, SEQ_PER_W)], wsem)
            pltpu.async_copy(
                planep, gpT_hbm.at[c, slice(None), pl.ds(s0, SEQ_PER_W)], wsem)

        pltpu.make_async_copy(
            planec, gcT_hbm.at[CTX - 1, slice(None), pl.ds(s0, SEQ_PER_W)],
            wsem).wait()
        pltpu.make_async_copy(
            planep, gpT_hbm.at[CTX - 1, slice(None), pl.ds(s0, SEQ_PER_W)],
            wsem).wait()

    return k(sentence, pos, char_table, pos_table)


def kernel(sentence, gazet, pos, words, char_table, pos_table):
    gcT, gpT = _sc_gather(sentence.astype(jnp.int32), pos.astype(jnp.int32),
                          char_table, pos_table)
    gc = jnp.transpose(gcT, (2, 0, 1))
    gp = jnp.transpose(gpT, (2, 0, 1))

    j = jnp.arange(1, CTX + 1, dtype=jnp.float32)[:, None]
    k = jnp.arange(1, EMB + 1, dtype=jnp.float32)[None, :]
    pe = 1.0 - j / CTX - (k / EMB) * (1.0 - 2.0 * j / CTX)

    return jnp.concatenate(
        [
            gc + pe[:, 0:CHAR_D],
            gp + pe[:, CHAR_D:CHAR_D + POS_D],
            words + pe[:, CHAR_D + POS_D:CHAR_D + POS_D + WORD_D],
            gazet + pe[:, CHAR_D + POS_D + WORD_D:EMB],
        ],
        axis=2,
    )
